# baseline (speedup 1.0000x reference)
import jax
import jax.numpy as jnp
from jax.experimental import pallas as pl
from jax.experimental.pallas import tpu as pltpu

_N = 10000
_FT = 128
_H = 64
_NS = 5
_BRS = 40
_ROWS_PER_STEP = _NS * _BRS
_G = _N // _ROWS_PER_STEP

_LAMBD = 0.001
_W = 1.0


def _body(x_ref, w10_ref, b10_ref, w11_ref, b11_ref,
          w20_ref, b20_ref, w21_ref, b21_ref,
          *refs):
    adj_refs = refs[:2 * _NS]
    out_ref = refs[2 * _NS]
    ha0_ref, ha1_ref, c0_ref, c1_ref, c01_ref = refs[2 * _NS + 1:]
    r = pl.program_id(0)

    @pl.when(r == 0)
    def _init():
        xv = x_ref[...]
        h0 = jnp.maximum(
            jnp.dot(xv, w10_ref[...], preferred_element_type=jnp.float32)
            + b10_ref[...], 0.0)
        ha0_ref[...] = (jnp.dot(h0, w11_ref[...],
                                preferred_element_type=jnp.float32)
                        + b11_ref[...])
        h1 = jnp.maximum(
            jnp.dot(xv, w20_ref[...], preferred_element_type=jnp.float32)
            + b20_ref[...], 0.0)
        ha1_ref[...] = (jnp.dot(h1, w21_ref[...],
                                preferred_element_type=jnp.float32)
                        + b21_ref[...])
        z = jnp.zeros((_H, _H), jnp.float32)
        c0_ref[...] = z
        c1_ref[...] = z
        c01_ref[...] = z

    dn = (((0,), (0,)), ((), ()))
    ha0 = ha0_ref[...]
    ha1 = ha1_ref[...]
    for s in range(_NS):
        base = r * _ROWS_PER_STEP + s * _BRS
        hp0 = jnp.dot(adj_refs[s][0], ha0,
                      preferred_element_type=jnp.float32)
        hp1 = jnp.dot(adj_refs[_NS + s][0], ha1,
                      preferred_element_type=jnp.float32)
        ha0r = ha0_ref[pl.ds(base, _BRS), :]
        ha1r = ha1_ref[pl.ds(base, _BRS), :]
        c0_ref[...] += jax.lax.dot_general(hp0, ha0r, dn,
                                           preferred_element_type=jnp.float32)
        c1_ref[...] += jax.lax.dot_general(hp1, ha1r, dn,
                                           preferred_element_type=jnp.float32)
        c01_ref[...] += jax.lax.dot_general(hp0, hp1, dn,
                                            preferred_element_type=jnp.float32)

    @pl.when(r == _G - 1)
    def _final():
        ri = jax.lax.broadcasted_iota(jnp.int32, (_H, _H), 0)
        ci = jax.lax.broadcasted_iota(jnp.int32, (_H, _H), 1)
        eye = ri == ci
        loss = jnp.float32(0.0)
        for c_ref in (c0_ref, c1_ref, c01_ref):
            cv = c_ref[...]
            sq = cv * cv
            on_diag = jnp.sum(jnp.where(eye, (cv - 1.0) ** 2, 0.0))
            off_diag = jnp.sum(sq) - jnp.sum(jnp.where(eye, sq, 0.0))
            loss = loss + (on_diag + _LAMBD * off_diag) * _W
        out_ref[...] = jnp.broadcast_to(loss, (1, 1))


def _adj_spec(view, sub):
    return pl.BlockSpec((1, _BRS, _N),
                        lambda r, v=view, s=sub: (v, r * _NS + s, 0))


def kernel(x, adj_list, W1_0, b1_0, W1_1, b1_1, W2_0, b2_0, W2_1, b2_1):
    const = lambda r: (0, 0)
    adj_specs = [_adj_spec(0, s) for s in range(_NS)] + \
                [_adj_spec(1, s) for s in range(_NS)]
    out = pl.pallas_call(
        _body,
        grid=(_G,),
        in_specs=[
            pl.BlockSpec((_N, _FT), const),
            pl.BlockSpec((_FT, _FT), const),
            pl.BlockSpec((1, _FT), const),
            pl.BlockSpec((_FT, _H), const),
            pl.BlockSpec((1, _H), const),
            pl.BlockSpec((_FT, _FT), const),
            pl.BlockSpec((1, _FT), const),
            pl.BlockSpec((_FT, _H), const),
            pl.BlockSpec((1, _H), const),
        ] + adj_specs,
        out_specs=pl.BlockSpec((1, 1), const),
        out_shape=jax.ShapeDtypeStruct((1, 1), jnp.float32),
        scratch_shapes=[
            pltpu.VMEM((_N, _H), jnp.float32),
            pltpu.VMEM((_N, _H), jnp.float32),
            pltpu.VMEM((_H, _H), jnp.float32),
            pltpu.VMEM((_H, _H), jnp.float32),
            pltpu.VMEM((_H, _H), jnp.float32),
        ],
    )(x, W1_0, b1_0.reshape(1, _FT), W1_1, b1_1.reshape(1, _H),
      W2_0, b2_0.reshape(1, _FT), W2_1, b2_1.reshape(1, _H),
      *([adj_list] * (2 * _NS)))
    loss = out[0, 0]
    return (loss, jnp.float32(0.0))


# speedup vs baseline: 1.0831x; 1.0831x over previous
import jax
import jax.numpy as jnp
from jax.experimental import pallas as pl
from jax.experimental.pallas import tpu as pltpu

_N = 10000
_FT = 128
_H = 64
_NS = 5
_BRS = 40
_ROWS_PER_STEP = _NS * _BRS
_G = _N // _ROWS_PER_STEP

_LAMBD = 0.001
_W = 1.0


def _body(x_ref, w10_ref, b10_ref, w11_ref, b11_ref,
          w20_ref, b20_ref, w21_ref, b21_ref,
          *refs):
    adj_refs = refs[:2 * _NS]
    out_ref = refs[2 * _NS]
    (ha0_ref, ha1_ref, hp0_ref, hp1_ref,
     c0_ref, c1_ref, c01_ref) = refs[2 * _NS + 1:]
    r = pl.program_id(0)

    @pl.when(r == 0)
    def _init():
        xv = x_ref[...]
        h0 = jnp.maximum(
            jnp.dot(xv, w10_ref[...], preferred_element_type=jnp.float32)
            + b10_ref[...], 0.0)
        ha0_ref[...] = (jnp.dot(h0, w11_ref[...],
                                preferred_element_type=jnp.float32)
                        + b11_ref[...])
        h1 = jnp.maximum(
            jnp.dot(xv, w20_ref[...], preferred_element_type=jnp.float32)
            + b20_ref[...], 0.0)
        ha1_ref[...] = (jnp.dot(h1, w21_ref[...],
                                preferred_element_type=jnp.float32)
                        + b21_ref[...])
        z = jnp.zeros((_H, _H), jnp.float32)
        c0_ref[...] = z
        c1_ref[...] = z
        c01_ref[...] = z

    dn = (((0,), (0,)), ((), ()))
    ha0 = ha0_ref[...]
    ha1 = ha1_ref[...]
    for s in range(_NS):
        hp0_ref[pl.ds(s * _BRS, _BRS), :] = jnp.dot(
            adj_refs[s][0], ha0, preferred_element_type=jnp.float32)
        hp1_ref[pl.ds(s * _BRS, _BRS), :] = jnp.dot(
            adj_refs[_NS + s][0], ha1, preferred_element_type=jnp.float32)
    hp0 = hp0_ref[...]
    hp1 = hp1_ref[...]
    base = r * _ROWS_PER_STEP
    ha0r = ha0_ref[pl.ds(base, _ROWS_PER_STEP), :]
    ha1r = ha1_ref[pl.ds(base, _ROWS_PER_STEP), :]
    c0_ref[...] += jax.lax.dot_general(hp0, ha0r, dn,
                                       preferred_element_type=jnp.float32)
    c1_ref[...] += jax.lax.dot_general(hp1, ha1r, dn,
                                       preferred_element_type=jnp.float32)
    c01_ref[...] += jax.lax.dot_general(hp0, hp1, dn,
                                        preferred_element_type=jnp.float32)

    @pl.when(r == _G - 1)
    def _final():
        ri = jax.lax.broadcasted_iota(jnp.int32, (_H, _H), 0)
        ci = jax.lax.broadcasted_iota(jnp.int32, (_H, _H), 1)
        eye = ri == ci
        loss = jnp.float32(0.0)
        for c_ref in (c0_ref, c1_ref, c01_ref):
            cv = c_ref[...]
            sq = cv * cv
            on_diag = jnp.sum(jnp.where(eye, (cv - 1.0) ** 2, 0.0))
            off_diag = jnp.sum(sq) - jnp.sum(jnp.where(eye, sq, 0.0))
            loss = loss + (on_diag + _LAMBD * off_diag) * _W
        out_ref[...] = jnp.broadcast_to(loss, (1, 1))


def _adj_spec(view, sub):
    return pl.BlockSpec((1, _BRS, _N),
                        lambda r, v=view, s=sub: (v, r * _NS + s, 0))


def kernel(x, adj_list, W1_0, b1_0, W1_1, b1_1, W2_0, b2_0, W2_1, b2_1):
    const = lambda r: (0, 0)
    adj_specs = [_adj_spec(0, s) for s in range(_NS)] + \
                [_adj_spec(1, s) for s in range(_NS)]
    out = pl.pallas_call(
        _body,
        grid=(_G,),
        in_specs=[
            pl.BlockSpec((_N, _FT), const),
            pl.BlockSpec((_FT, _FT), const),
            pl.BlockSpec((1, _FT), const),
            pl.BlockSpec((_FT, _H), const),
            pl.BlockSpec((1, _H), const),
            pl.BlockSpec((_FT, _FT), const),
            pl.BlockSpec((1, _FT), const),
            pl.BlockSpec((_FT, _H), const),
            pl.BlockSpec((1, _H), const),
        ] + adj_specs,
        out_specs=pl.BlockSpec((1, 1), const),
        out_shape=jax.ShapeDtypeStruct((1, 1), jnp.float32),
        scratch_shapes=[
            pltpu.VMEM((_N, _H), jnp.float32),
            pltpu.VMEM((_N, _H), jnp.float32),
            pltpu.VMEM((_ROWS_PER_STEP, _H), jnp.float32),
            pltpu.VMEM((_ROWS_PER_STEP, _H), jnp.float32),
            pltpu.VMEM((_H, _H), jnp.float32),
            pltpu.VMEM((_H, _H), jnp.float32),
            pltpu.VMEM((_H, _H), jnp.float32),
        ],
    )(x, W1_0, b1_0.reshape(1, _FT), W1_1, b1_1.reshape(1, _H),
      W2_0, b2_0.reshape(1, _FT), W2_1, b2_1.reshape(1, _H),
      *([adj_list] * (2 * _NS)))
    loss = out[0, 0]
    return (loss, jnp.float32(0.0))


# speedup vs baseline: 1.1058x; 1.0210x over previous
import jax
import jax.numpy as jnp
from jax.experimental import pallas as pl
from jax.experimental.pallas import tpu as pltpu

_N = 10000
_FT = 128
_H = 64
_NS = 1
_BRS = 200
_ROWS_PER_STEP = _NS * _BRS
_G = _N // _ROWS_PER_STEP

_LAMBD = 0.001
_W = 1.0


def _body(x_ref, w10_ref, b10_ref, w11_ref, b11_ref,
          w20_ref, b20_ref, w21_ref, b21_ref,
          *refs):
    adj_refs = refs[:2 * _NS]
    out_ref = refs[2 * _NS]
    (ha0_ref, ha1_ref, hp0_ref, hp1_ref,
     c0_ref, c1_ref, c01_ref) = refs[2 * _NS + 1:]
    r = pl.program_id(0)

    @pl.when(r == 0)
    def _init():
        xv = x_ref[...]
        h0 = jnp.maximum(
            jnp.dot(xv, w10_ref[...], preferred_element_type=jnp.float32)
            + b10_ref[...], 0.0)
        ha0_ref[...] = (jnp.dot(h0, w11_ref[...],
                                preferred_element_type=jnp.float32)
                        + b11_ref[...])
        h1 = jnp.maximum(
            jnp.dot(xv, w20_ref[...], preferred_element_type=jnp.float32)
            + b20_ref[...], 0.0)
        ha1_ref[...] = (jnp.dot(h1, w21_ref[...],
                                preferred_element_type=jnp.float32)
                        + b21_ref[...])
        z = jnp.zeros((_H, _H), jnp.float32)
        c0_ref[...] = z
        c1_ref[...] = z
        c01_ref[...] = z

    dn = (((0,), (0,)), ((), ()))
    ha0 = ha0_ref[...]
    ha1 = ha1_ref[...]
    if _NS == 1:
        hp0 = jnp.dot(adj_refs[0][0], ha0,
                      preferred_element_type=jnp.float32)
        hp1 = jnp.dot(adj_refs[1][0], ha1,
                      preferred_element_type=jnp.float32)
    else:
        for s in range(_NS):
            hp0_ref[pl.ds(s * _BRS, _BRS), :] = jnp.dot(
                adj_refs[s][0], ha0, preferred_element_type=jnp.float32)
            hp1_ref[pl.ds(s * _BRS, _BRS), :] = jnp.dot(
                adj_refs[_NS + s][0], ha1, preferred_element_type=jnp.float32)
        hp0 = hp0_ref[...]
        hp1 = hp1_ref[...]
    base = r * _ROWS_PER_STEP
    ha0r = ha0_ref[pl.ds(base, _ROWS_PER_STEP), :]
    ha1r = ha1_ref[pl.ds(base, _ROWS_PER_STEP), :]
    c0_ref[...] += jax.lax.dot_general(hp0, ha0r, dn,
                                       preferred_element_type=jnp.float32)
    c1_ref[...] += jax.lax.dot_general(hp1, ha1r, dn,
                                       preferred_element_type=jnp.float32)
    c01_ref[...] += jax.lax.dot_general(hp0, hp1, dn,
                                        preferred_element_type=jnp.float32)

    @pl.when(r == _G - 1)
    def _final():
        ri = jax.lax.broadcasted_iota(jnp.int32, (_H, _H), 0)
        ci = jax.lax.broadcasted_iota(jnp.int32, (_H, _H), 1)
        eye = ri == ci
        loss = jnp.float32(0.0)
        for c_ref in (c0_ref, c1_ref, c01_ref):
            cv = c_ref[...]
            sq = cv * cv
            on_diag = jnp.sum(jnp.where(eye, (cv - 1.0) ** 2, 0.0))
            off_diag = jnp.sum(sq) - jnp.sum(jnp.where(eye, sq, 0.0))
            loss = loss + (on_diag + _LAMBD * off_diag) * _W
        out_ref[...] = jnp.broadcast_to(loss, (1, 1))


def _adj_spec(view, sub):
    return pl.BlockSpec((1, _BRS, _N),
                        lambda r, v=view, s=sub: (v, r * _NS + s, 0))


def kernel(x, adj_list, W1_0, b1_0, W1_1, b1_1, W2_0, b2_0, W2_1, b2_1):
    const = lambda r: (0, 0)
    adj_specs = [_adj_spec(0, s) for s in range(_NS)] + \
                [_adj_spec(1, s) for s in range(_NS)]
    out = pl.pallas_call(
        _body,
        grid=(_G,),
        in_specs=[
            pl.BlockSpec((_N, _FT), const),
            pl.BlockSpec((_FT, _FT), const),
            pl.BlockSpec((1, _FT), const),
            pl.BlockSpec((_FT, _H), const),
            pl.BlockSpec((1, _H), const),
            pl.BlockSpec((_FT, _FT), const),
            pl.BlockSpec((1, _FT), const),
            pl.BlockSpec((_FT, _H), const),
            pl.BlockSpec((1, _H), const),
        ] + adj_specs,
        out_specs=pl.BlockSpec((1, 1), const),
        out_shape=jax.ShapeDtypeStruct((1, 1), jnp.float32),
        scratch_shapes=[
            pltpu.VMEM((_N, _H), jnp.float32),
            pltpu.VMEM((_N, _H), jnp.float32),
            pltpu.VMEM((_ROWS_PER_STEP, _H), jnp.float32),
            pltpu.VMEM((_ROWS_PER_STEP, _H), jnp.float32),
            pltpu.VMEM((_H, _H), jnp.float32),
            pltpu.VMEM((_H, _H), jnp.float32),
            pltpu.VMEM((_H, _H), jnp.float32),
        ],
    )(x, W1_0, b1_0.reshape(1, _FT), W1_1, b1_1.reshape(1, _H),
      W2_0, b2_0.reshape(1, _FT), W2_1, b2_1.reshape(1, _H),
      *([adj_list] * (2 * _NS)))
    loss = out[0, 0]
    return (loss, jnp.float32(0.0))


# speedup vs baseline: 1.1233x; 1.0158x over previous
import jax
import jax.numpy as jnp
from jax.experimental import pallas as pl
from jax.experimental.pallas import tpu as pltpu

_N = 10000
_FT = 128
_H = 64
_BR = 200
_NRB = _N // _BR

_LAMBD = 0.001
_W = 1.0


def _body(x_ref, w10_ref, b10_ref, w11_ref, b11_ref,
          w20_ref, b20_ref, w21_ref, b21_ref,
          adj0_ref, adj1_ref,
          out_ref,
          ha0_ref, ha1_ref, c0_ref, c1_ref, c01_ref):
    r = pl.program_id(0)

    @pl.when(r == 0)
    def _init():
        xv = x_ref[...]
        h0 = jnp.maximum(
            jnp.dot(xv, w10_ref[...], preferred_element_type=jnp.float32)
            + b10_ref[...], 0.0)
        ha0_ref[...] = (jnp.dot(h0, w11_ref[...],
                                preferred_element_type=jnp.float32)
                        + b11_ref[...])
        h1 = jnp.maximum(
            jnp.dot(xv, w20_ref[...], preferred_element_type=jnp.float32)
            + b20_ref[...], 0.0)
        ha1_ref[...] = (jnp.dot(h1, w21_ref[...],
                                preferred_element_type=jnp.float32)
                        + b21_ref[...])
        z = jnp.zeros((_H, _H), jnp.float32)
        c0_ref[...] = z
        c1_ref[...] = z
        c01_ref[...] = z

    a0 = adj0_ref[0]
    a1 = adj1_ref[0]
    hp0 = jnp.dot(a0, ha0_ref[...], preferred_element_type=jnp.float32)
    hp1 = jnp.dot(a1, ha1_ref[...], preferred_element_type=jnp.float32)
    ha0r = ha0_ref[pl.ds(r * _BR, _BR), :]
    ha1r = ha1_ref[pl.ds(r * _BR, _BR), :]
    dn = (((0,), (0,)), ((), ()))
    c0_ref[...] += jax.lax.dot_general(hp0, ha0r, dn,
                                       preferred_element_type=jnp.float32)
    c1_ref[...] += jax.lax.dot_general(hp1, ha1r, dn,
                                       preferred_element_type=jnp.float32)
    c01_ref[...] += jax.lax.dot_general(hp0, hp1, dn,
                                        preferred_element_type=jnp.float32)

    @pl.when(r == _NRB - 1)
    def _final():
        ri = jax.lax.broadcasted_iota(jnp.int32, (_H, _H), 0)
        ci = jax.lax.broadcasted_iota(jnp.int32, (_H, _H), 1)
        eye = ri == ci
        loss = jnp.float32(0.0)
        for c_ref in (c0_ref, c1_ref, c01_ref):
            cv = c_ref[...]
            sq = cv * cv
            on_diag = jnp.sum(jnp.where(eye, (cv - 1.0) ** 2, 0.0))
            off_diag = jnp.sum(sq) - jnp.sum(jnp.where(eye, sq, 0.0))
            loss = loss + (on_diag + _LAMBD * off_diag) * _W
        out_ref[...] = jnp.broadcast_to(loss, (1, 1))


def kernel(x, adj_list, W1_0, b1_0, W1_1, b1_1, W2_0, b2_0, W2_1, b2_1):
    const = lambda r: (0, 0)
    out = pl.pallas_call(
        _body,
        grid=(_NRB,),
        in_specs=[
            pl.BlockSpec((_N, _FT), const),
            pl.BlockSpec((_FT, _FT), const),
            pl.BlockSpec((1, _FT), const),
            pl.BlockSpec((_FT, _H), const),
            pl.BlockSpec((1, _H), const),
            pl.BlockSpec((_FT, _FT), const),
            pl.BlockSpec((1, _FT), const),
            pl.BlockSpec((_FT, _H), const),
            pl.BlockSpec((1, _H), const),
            pl.BlockSpec((1, _BR, _N), lambda r: (0, r, 0)),
            pl.BlockSpec((1, _BR, _N), lambda r: (1, r, 0)),
        ],
        out_specs=pl.BlockSpec((1, 1), const),
        out_shape=jax.ShapeDtypeStruct((1, 1), jnp.float32),
        scratch_shapes=[
            pltpu.VMEM((_N, _H), jnp.float32),
            pltpu.VMEM((_N, _H), jnp.float32),
            pltpu.VMEM((_H, _H), jnp.float32),
            pltpu.VMEM((_H, _H), jnp.float32),
            pltpu.VMEM((_H, _H), jnp.float32),
        ],
    )(x, W1_0, b1_0.reshape(1, _FT), W1_1, b1_1.reshape(1, _H),
      W2_0, b2_0.reshape(1, _FT), W2_1, b2_1.reshape(1, _H),
      adj_list, adj_list)
    loss = out[0, 0]
    return (loss, jnp.float32(0.0))
